# baseline (speedup 1.0000x reference)
import jax
import jax.numpy as jnp
from jax import lax
from jax.experimental import pallas as pl
from jax.experimental.pallas import tpu as pltpu

_K = 4


def _body(idx_ref, tbl_ref, out_ref, buf_ref, sem_ref):
    i = pl.program_id(0)
    n = pl.num_programs(0)
    dim, vocab = tbl_ref.shape
    hw = buf_ref.shape[3]
    p = lax.rem(i, 2)

    for k in range(_K):
        @pl.when(i >= 2)
        def _():
            pltpu.make_async_copy(
                buf_ref.at[p, k], out_ref.at[(i - 2) * _K + k], sem_ref.at[p, k]
            ).wait()

        f = i * _K + k
        v = idx_ref[f]
        sel = lax.broadcasted_iota(jnp.int32, (dim, vocab), 1) == v
        col = jnp.sum(jnp.where(sel, tbl_ref[...], 0.0), axis=1, keepdims=True)
        buf_ref[p, k] = jnp.broadcast_to(col, (dim, hw))
        pltpu.make_async_copy(
            buf_ref.at[p, k], out_ref.at[f], sem_ref.at[p, k]
        ).start()

    @pl.when(i == n - 1)
    def _():
        for k in range(_K):
            pltpu.make_async_copy(
                buf_ref.at[1 - p, k], out_ref.at[(n - 2) * _K + k],
                sem_ref.at[1 - p, k],
            ).wait()
            pltpu.make_async_copy(
                buf_ref.at[p, k], out_ref.at[(n - 1) * _K + k], sem_ref.at[p, k]
            ).wait()


def kernel(spatialPos, numFrames, frameIndices, frameEmbed):
    _, _, height, width = spatialPos.shape
    n_frames = frameIndices.shape[0]
    vocab, dim = frameEmbed.shape
    hw = height * width

    grid_spec = pltpu.PrefetchScalarGridSpec(
        num_scalar_prefetch=1,
        grid=(n_frames // _K,),
        in_specs=[pl.BlockSpec((dim, vocab), lambda i, s: (0, 0))],
        out_specs=pl.BlockSpec(memory_space=pltpu.MemorySpace.HBM),
        scratch_shapes=[
            pltpu.VMEM((2, _K, dim, hw), jnp.float32),
            pltpu.SemaphoreType.DMA((2, _K)),
        ],
    )
    out = pl.pallas_call(
        _body,
        grid_spec=grid_spec,
        out_shape=jax.ShapeDtypeStruct((n_frames, dim, hw), jnp.float32),
    )(frameIndices.astype(jnp.int32), frameEmbed.T)

    return out.reshape(n_frames, dim, height, width)


# speedup vs baseline: 1.0219x; 1.0219x over previous
import jax
import jax.numpy as jnp
from jax import lax
from jax.experimental import pallas as pl
from jax.experimental.pallas import tpu as pltpu

_K = 8


def _body(idx_ref, tbl_ref, out_ref, *scratch):
    bufs, sems = scratch[:_K], scratch[_K:]
    i = pl.program_id(0)
    n = pl.num_programs(0)
    dim, vocab = tbl_ref.shape
    hw = bufs[0].shape[1]

    for k in range(_K):
        @pl.when(i > 0)
        def _():
            pltpu.make_async_copy(
                bufs[k], out_ref.at[(i - 1) * _K + k], sems[k]
            ).wait()

        f = i * _K + k
        v = idx_ref[f]
        sel = lax.broadcasted_iota(jnp.int32, (dim, vocab), 1) == v
        col = jnp.sum(jnp.where(sel, tbl_ref[...], 0.0), axis=1, keepdims=True)
        bufs[k][...] = jnp.broadcast_to(col, (dim, hw))
        pltpu.make_async_copy(bufs[k], out_ref.at[f], sems[k]).start()

    @pl.when(i == n - 1)
    def _():
        for k in range(_K):
            pltpu.make_async_copy(
                bufs[k], out_ref.at[(n - 1) * _K + k], sems[k]
            ).wait()


def kernel(spatialPos, numFrames, frameIndices, frameEmbed):
    _, _, height, width = spatialPos.shape
    n_frames = frameIndices.shape[0]
    vocab, dim = frameEmbed.shape
    hw = height * width

    grid_spec = pltpu.PrefetchScalarGridSpec(
        num_scalar_prefetch=1,
        grid=(n_frames // _K,),
        in_specs=[pl.BlockSpec((dim, vocab), lambda i, s: (0, 0))],
        out_specs=pl.BlockSpec(memory_space=pltpu.MemorySpace.HBM),
        scratch_shapes=(
            [pltpu.VMEM((dim, hw), jnp.float32) for _ in range(_K)]
            + [pltpu.SemaphoreType.DMA for _ in range(_K)]
        ),
    )
    out = pl.pallas_call(
        _body,
        grid_spec=grid_spec,
        out_shape=jax.ShapeDtypeStruct((n_frames, dim, hw), jnp.float32),
    )(frameIndices.astype(jnp.int32), frameEmbed.T)

    return out.reshape(n_frames, dim, height, width)
